# baseline (speedup 1.0000x reference)
import jax
import jax.numpy as jnp
from jax import lax
from jax.experimental import pallas as pl
from jax.experimental.pallas import tpu as pltpu
from jax.experimental.pallas import tpu_sc as plsc

N = 10000
E = 320000
D = 128
D_E = 16
NC, NS = 2, 16
NW = NC * NS
CH = 128
NCHUNK = 80
SCH = 64
NSTREAM = NCHUNK * (CH // SCH)
NGRP = NSTREAM // 4
EPAD = NW * NCHUNK * CH - E
RPT = 624
RPT_LAST = N - RPT * (NS - 1)
RB = 2000


def _prep_body(vfeat_ref, w_ref, b_ref, dv2_ref, out_ref):
    wh = jnp.dot(vfeat_ref[...], w_ref[...], preferred_element_type=jnp.float32)
    out_ref[...] = (wh + b_ref[...]) * dv2_ref[...]


_prep = pl.pallas_call(
    _prep_body,
    grid=(N // RB,),
    in_specs=[
        pl.BlockSpec((RB, D), lambda i: (i, 0)),
        pl.BlockSpec((D, D), lambda i: (0, 0)),
        pl.BlockSpec((1, D), lambda i: (0, 0)),
        pl.BlockSpec((RB, 1), lambda i: (i, 0)),
    ],
    out_specs=pl.BlockSpec((RB, D), lambda i: (i, 0)),
    out_shape=jax.ShapeDtypeStruct((N, D), jnp.float32),
)


def _mid_body(p_ref, inv_ref, we_ref, be_ref, e2_ref, eout_ref):
    en = p_ref[0] + p_ref[1]
    e2_ref[...] = en * inv_ref[...]
    eout_ref[...] = (
        jnp.dot(en, we_ref[...], preferred_element_type=jnp.float32) + be_ref[...]
    )


_mid = pl.pallas_call(
    _mid_body,
    grid=(N // RB,),
    in_specs=[
        pl.BlockSpec((2, RB, D), lambda i: (0, i, 0)),
        pl.BlockSpec((RB, 1), lambda i: (i, 0)),
        pl.BlockSpec((D, D_E), lambda i: (0, 0)),
        pl.BlockSpec((1, D_E), lambda i: (0, 0)),
    ],
    out_specs=[
        pl.BlockSpec((RB, D), lambda i: (i, 0)),
        pl.BlockSpec((RB, D_E), lambda i: (i, 0)),
    ],
    out_shape=[
        jax.ShapeDtypeStruct((N, D), jnp.float32),
        jax.ShapeDtypeStruct((N, D_E), jnp.float32),
    ],
)


def _final_body(s_ref, dv2_ref, out_ref):
    out_ref[...] = jnp.maximum((s_ref[0] + s_ref[1]) * dv2_ref[...], 0.0)


_final = pl.pallas_call(
    _final_body,
    grid=(N // RB,),
    in_specs=[
        pl.BlockSpec((2, RB, D), lambda i: (0, i, 0)),
        pl.BlockSpec((RB, 1), lambda i: (i, 0)),
    ],
    out_specs=pl.BlockSpec((RB, D), lambda i: (i, 0)),
    out_shape=jax.ShapeDtypeStruct((N, D), jnp.float32),
)


def _sc_body(
    table, pidx, out,
    pk_v, g0, g1, g2, g3, s0, s1, s2, s3, b0, b1, b2, b3, acc,
    gm0, gm1, gm2, gm3, sm0, sm1, sm2, sm3,
):
    gs = (g0, g1, g2, g3)
    ss = (s0, s1, s2, s3)
    bs = (b0, b1, b2, b3)
    gms = (gm0, gm1, gm2, gm3)
    sms = (sm0, sm1, sm2, sm3)
    cid = lax.axis_index("c")
    sid = lax.axis_index("s")
    w = sid * NC + cid
    start = pl.multiple_of(sid * RPT, 8)
    pltpu.sync_copy(pidx.at[w], pk_v)

    def unpack(row, half, gbuf, sbuf):
        for j in range(SCH // 16):
            v = pk_v[row, pl.ds(half * SCH + j * 16, 16)]
            gbuf[pl.ds(j * 16, 16)] = v & 0xFFFF
            sbuf[pl.ds(j * 16, 16)] = lax.shift_right_logical(v, 16)

    def zrow(i, carry):
        for j in range(D // 16):
            b0[i, pl.ds(j * 16, 16)] = jnp.zeros((16,), jnp.float32)
        return carry

    lax.fori_loop(0, SCH, zrow, 0)

    @pl.when(sid < NS - 1)
    def _():
        for k in range(RPT // SCH):
            pltpu.sync_copy(b0, acc.at[pl.ds(start + k * SCH, SCH), :])
        pltpu.sync_copy(
            b0.at[pl.ds(0, RPT % SCH)],
            acc.at[pl.ds(start + (RPT // SCH) * SCH, RPT % SCH), :],
        )

    @pl.when(sid == NS - 1)
    def _():
        for k in range(RPT_LAST // SCH):
            pltpu.sync_copy(b0, acc.at[pl.ds(start + k * SCH, SCH), :])

    plsc.subcore_barrier()

    for k in range(4):
        unpack(k // 2, k % 2, gs[k], ss[k])
        pltpu.async_copy(table.at[gs[k]], bs[k], gms[k])
        if k >= 1:
            pltpu.make_async_copy(table.at[gs[k - 1]], bs[k - 1], gms[k - 1]).wait()
            pltpu.async_copy(bs[k - 1], acc.at[ss[k - 1]], sms[k - 1], add=True)

    def group(g, carry):
        row = 2 * g
        for k in range(4):
            km1 = (k - 1) % 4
            pltpu.make_async_copy(bs[k], acc.at[ss[k]], sms[k]).wait()
            unpack(row + k // 2, k % 2, gs[k], ss[k])
            pltpu.async_copy(table.at[gs[k]], bs[k], gms[k])
            pltpu.make_async_copy(table.at[gs[km1]], bs[km1], gms[km1]).wait()
            pltpu.async_copy(bs[km1], acc.at[ss[km1]], sms[km1], add=True)
        return carry

    lax.fori_loop(1, NGRP, group, 0)
    pltpu.make_async_copy(table.at[gs[3]], bs[3], gms[3]).wait()
    pltpu.async_copy(bs[3], acc.at[ss[3]], sms[3], add=True)
    for k in range(4):
        pltpu.make_async_copy(bs[k], acc.at[ss[k]], sms[k]).wait()
    plsc.subcore_barrier()

    @pl.when(sid < NS - 1)
    def _():
        pltpu.sync_copy(
            acc.at[pl.ds(start, RPT), :], out.at[cid, pl.ds(start, RPT), :]
        )

    @pl.when(sid == NS - 1)
    def _():
        pltpu.sync_copy(
            acc.at[pl.ds(start, RPT_LAST), :],
            out.at[cid, pl.ds(start, RPT_LAST), :],
        )


_sc_scatter = pl.kernel(
    _sc_body,
    out_type=jax.ShapeDtypeStruct((NC, N, D), jnp.float32),
    mesh=plsc.VectorSubcoreMesh(
        core_axis_name="c", subcore_axis_name="s", num_cores=NC, num_subcores=NS
    ),
    scratch_types=(
        [pltpu.VMEM((NCHUNK, CH), jnp.int32)]
        + [pltpu.VMEM((SCH,), jnp.int32)] * 8
        + [pltpu.VMEM((SCH, D), jnp.float32)] * 4
        + [pltpu.VMEM_SHARED((N, D), jnp.float32)]
        + [pltpu.SemaphoreType.DMA] * 8
    ),
)


def kernel(vfeat, efeat, DV2, invDE, edge_index, W_v, b_v, W_e, b_e):
    src = edge_index[0]
    dst = edge_index[1]
    ept = NCHUNK * CH - E // NW
    src2 = src.reshape(NW, E // NW)
    dst2 = dst.reshape(NW, E // NW)
    gpad = jnp.broadcast_to(
        N + jnp.arange(ept, dtype=jnp.int32), (NW, ept)
    )
    spad = (
        jnp.arange(ept, dtype=jnp.int32)[None, :]
        + ept * jnp.arange(NW, dtype=jnp.int32)[:, None]
    ) % N
    pk1 = (
        jnp.concatenate([src2, gpad], axis=1)
        | (jnp.concatenate([dst2, spad], axis=1) << 16)
    ).reshape(NW, NCHUNK, CH)
    pk2 = (
        jnp.concatenate([dst2, gpad], axis=1)
        | (jnp.concatenate([src2, spad], axis=1) << 16)
    ).reshape(NW, NCHUNK, CH)
    zrows = jnp.zeros((ept + 8, D), jnp.float32)
    dv2c = DV2.reshape(N, 1)
    t1 = _prep(vfeat, W_v, b_v.reshape(1, D), dv2c)
    p = _sc_scatter(jnp.concatenate([t1, zrows]), pk1)
    e2, efeat_out = _mid(p, invDE.reshape(N, 1), W_e, b_e.reshape(1, D_E))
    s = _sc_scatter(jnp.concatenate([e2, zrows]), pk2)
    vfeat_out = _final(s, dv2c)
    return (vfeat_out, efeat_out)


# speedup vs baseline: 1.0121x; 1.0121x over previous
import jax
import jax.numpy as jnp
from jax import lax
from jax.experimental import pallas as pl
from jax.experimental.pallas import tpu as pltpu
from jax.experimental.pallas import tpu_sc as plsc

N = 10000
E = 320000
D = 128
D_E = 16
NC, NS = 2, 16
NW = NC * NS
CH = 128
NCHUNK = 80
SCH = 64
NSTREAM = NCHUNK * (CH // SCH)
NGRP = NSTREAM // 4
EPAD = NW * NCHUNK * CH - E
RPT = 624
RPT_LAST = N - RPT * (NS - 1)
RB = 2000


def _prep_body(vfeat_ref, w_ref, b_ref, dv2_ref, out_ref):
    wh = jnp.dot(vfeat_ref[...], w_ref[...], preferred_element_type=jnp.float32)
    out_ref[...] = (wh + b_ref[...]) * dv2_ref[...]


_prep = pl.pallas_call(
    _prep_body,
    grid=(N // RB,),
    in_specs=[
        pl.BlockSpec((RB, D), lambda i: (i, 0)),
        pl.BlockSpec((D, D), lambda i: (0, 0)),
        pl.BlockSpec((1, D), lambda i: (0, 0)),
        pl.BlockSpec((RB, 1), lambda i: (i, 0)),
    ],
    out_specs=pl.BlockSpec((RB, D), lambda i: (i, 0)),
    out_shape=jax.ShapeDtypeStruct((N, D), jnp.float32),
)


def _mid_body(p_ref, inv_ref, we_ref, be_ref, e2_ref, eout_ref):
    en = p_ref[0] + p_ref[1]
    e2_ref[...] = en * inv_ref[...]
    eout_ref[...] = (
        jnp.dot(en, we_ref[...], preferred_element_type=jnp.float32) + be_ref[...]
    )


_mid = pl.pallas_call(
    _mid_body,
    grid=(N // RB,),
    in_specs=[
        pl.BlockSpec((2, RB, D), lambda i: (0, i, 0)),
        pl.BlockSpec((RB, 1), lambda i: (i, 0)),
        pl.BlockSpec((D, D_E), lambda i: (0, 0)),
        pl.BlockSpec((1, D_E), lambda i: (0, 0)),
    ],
    out_specs=[
        pl.BlockSpec((RB, D), lambda i: (i, 0)),
        pl.BlockSpec((RB, D_E), lambda i: (i, 0)),
    ],
    out_shape=[
        jax.ShapeDtypeStruct((N, D), jnp.float32),
        jax.ShapeDtypeStruct((N, D_E), jnp.float32),
    ],
)


def _final_body(s_ref, dv2_ref, out_ref):
    out_ref[...] = jnp.maximum((s_ref[0] + s_ref[1]) * dv2_ref[...], 0.0)


_final = pl.pallas_call(
    _final_body,
    grid=(N // RB,),
    in_specs=[
        pl.BlockSpec((2, RB, D), lambda i: (0, i, 0)),
        pl.BlockSpec((RB, 1), lambda i: (i, 0)),
    ],
    out_specs=pl.BlockSpec((RB, D), lambda i: (i, 0)),
    out_shape=jax.ShapeDtypeStruct((N, D), jnp.float32),
)


EPN = E // NW
EPT = NCHUNK * CH - EPN
TPW = NCHUNK * CH


def _pack_body(ei_ref, pk1_ref, pk2_ref):
    s = ei_ref[0]
    d = ei_ref[1]
    pk1_ref[:, :EPN] = s | (d << 16)
    pk2_ref[:, :EPN] = d | (s << 16)
    cols = lax.broadcasted_iota(jnp.int32, (NW, EPT), 1)
    wv = lax.broadcasted_iota(jnp.int32, (NW, EPT), 0)
    pad = (N + cols) | (((cols + wv * EPT) % N) << 16)
    pk1_ref[:, EPN:] = pad
    pk2_ref[:, EPN:] = pad


_pack = pl.pallas_call(
    _pack_body,
    out_shape=[
        jax.ShapeDtypeStruct((NW, TPW), jnp.int32),
        jax.ShapeDtypeStruct((NW, TPW), jnp.int32),
    ],
)


def _sc_body(
    table, pidx, out,
    pk_v, g0, g1, g2, g3, s0, s1, s2, s3, b0, b1, b2, b3, acc,
    gm0, gm1, gm2, gm3, sm0, sm1, sm2, sm3,
):
    gs = (g0, g1, g2, g3)
    ss = (s0, s1, s2, s3)
    bs = (b0, b1, b2, b3)
    gms = (gm0, gm1, gm2, gm3)
    sms = (sm0, sm1, sm2, sm3)
    cid = lax.axis_index("c")
    sid = lax.axis_index("s")
    w = sid * NC + cid
    start = pl.multiple_of(sid * RPT, 8)
    pltpu.sync_copy(pidx.at[w], pk_v)

    def unpack(row, half, gbuf, sbuf):
        for j in range(SCH // 16):
            v = pk_v[row, pl.ds(half * SCH + j * 16, 16)]
            gbuf[pl.ds(j * 16, 16)] = v & 0xFFFF
            sbuf[pl.ds(j * 16, 16)] = lax.shift_right_logical(v, 16)

    def zrow(i, carry):
        for j in range(D // 16):
            b0[i, pl.ds(j * 16, 16)] = jnp.zeros((16,), jnp.float32)
        return carry

    lax.fori_loop(0, SCH, zrow, 0)

    @pl.when(sid < NS - 1)
    def _():
        for k in range(RPT // SCH):
            pltpu.sync_copy(b0, acc.at[pl.ds(start + k * SCH, SCH), :])
        pltpu.sync_copy(
            b0.at[pl.ds(0, RPT % SCH)],
            acc.at[pl.ds(start + (RPT // SCH) * SCH, RPT % SCH), :],
        )

    @pl.when(sid == NS - 1)
    def _():
        for k in range(RPT_LAST // SCH):
            pltpu.sync_copy(b0, acc.at[pl.ds(start + k * SCH, SCH), :])

    plsc.subcore_barrier()

    for k in range(4):
        unpack(k // 2, k % 2, gs[k], ss[k])
        pltpu.async_copy(table.at[gs[k]], bs[k], gms[k])
        if k >= 1:
            pltpu.make_async_copy(table.at[gs[k - 1]], bs[k - 1], gms[k - 1]).wait()
            pltpu.async_copy(bs[k - 1], acc.at[ss[k - 1]], sms[k - 1], add=True)

    def group(g, carry):
        row = 2 * g
        for k in range(4):
            km1 = (k - 1) % 4
            pltpu.make_async_copy(bs[k], acc.at[ss[k]], sms[k]).wait()
            unpack(row + k // 2, k % 2, gs[k], ss[k])
            pltpu.async_copy(table.at[gs[k]], bs[k], gms[k])
            pltpu.make_async_copy(table.at[gs[km1]], bs[km1], gms[km1]).wait()
            pltpu.async_copy(bs[km1], acc.at[ss[km1]], sms[km1], add=True)
        return carry

    lax.fori_loop(1, NGRP, group, 0)
    pltpu.make_async_copy(table.at[gs[3]], bs[3], gms[3]).wait()
    pltpu.async_copy(bs[3], acc.at[ss[3]], sms[3], add=True)
    for k in range(4):
        pltpu.make_async_copy(bs[k], acc.at[ss[k]], sms[k]).wait()
    plsc.subcore_barrier()

    @pl.when(sid < NS - 1)
    def _():
        pltpu.sync_copy(
            acc.at[pl.ds(start, RPT), :], out.at[cid, pl.ds(start, RPT), :]
        )

    @pl.when(sid == NS - 1)
    def _():
        pltpu.sync_copy(
            acc.at[pl.ds(start, RPT_LAST), :],
            out.at[cid, pl.ds(start, RPT_LAST), :],
        )


_sc_scatter = pl.kernel(
    _sc_body,
    out_type=jax.ShapeDtypeStruct((NC, N, D), jnp.float32),
    mesh=plsc.VectorSubcoreMesh(
        core_axis_name="c", subcore_axis_name="s", num_cores=NC, num_subcores=NS
    ),
    scratch_types=(
        [pltpu.VMEM((NCHUNK, CH), jnp.int32)]
        + [pltpu.VMEM((SCH,), jnp.int32)] * 8
        + [pltpu.VMEM((SCH, D), jnp.float32)] * 4
        + [pltpu.VMEM_SHARED((N, D), jnp.float32)]
        + [pltpu.SemaphoreType.DMA] * 8
    ),
)


def kernel(vfeat, efeat, DV2, invDE, edge_index, W_v, b_v, W_e, b_e):
    src = edge_index[0]
    dst = edge_index[1]
    pk1f, pk2f = _pack(edge_index.reshape(2, NW, EPN))
    pk1 = pk1f.reshape(NW, NCHUNK, CH)
    pk2 = pk2f.reshape(NW, NCHUNK, CH)
    zrows = jnp.zeros((EPT + 8, D), jnp.float32)
    dv2c = DV2.reshape(N, 1)
    t1 = _prep(vfeat, W_v, b_v.reshape(1, D), dv2c)
    p = _sc_scatter(jnp.concatenate([t1, zrows]), pk1)
    e2, efeat_out = _mid(p, invDE.reshape(N, 1), W_e, b_e.reshape(1, D_E))
    s = _sc_scatter(jnp.concatenate([e2, zrows]), pk2)
    vfeat_out = _final(s, dv2c)
    return (vfeat_out, efeat_out)


# speedup vs baseline: 1.0447x; 1.0322x over previous
import jax
import jax.numpy as jnp
from jax import lax
from jax.experimental import pallas as pl
from jax.experimental.pallas import tpu as pltpu
from jax.experimental.pallas import tpu_sc as plsc

N = 10000
E = 320000
D = 128
D_E = 16
NC, NS = 2, 16
NW = NC * NS
CH = 128
NCHUNK = 80
SCH = 64
NSTREAM = NCHUNK * (CH // SCH)
NGRP = NSTREAM // 4
EPAD = NW * NCHUNK * CH - E
RPT = 624
RPT_LAST = N - RPT * (NS - 1)
RB = 2000
TEXTRA = 400
TROWS = N + TEXTRA
RB2 = TROWS // 5


def _prep_body(vfeat_ref, w_ref, b_ref, dv2_ref, out_ref):
    i = pl.program_id(0)
    wh = jnp.dot(vfeat_ref[...], w_ref[...], preferred_element_type=jnp.float32)
    rows = i * RB2 + lax.broadcasted_iota(jnp.int32, (RB2, 1), 0)
    out_ref[...] = jnp.where(rows < N, (wh + b_ref[...]) * dv2_ref[...], 0.0)


_prep = pl.pallas_call(
    _prep_body,
    grid=(5,),
    in_specs=[
        pl.BlockSpec((RB2, D), lambda i: (i, 0)),
        pl.BlockSpec((D, D), lambda i: (0, 0)),
        pl.BlockSpec((1, D), lambda i: (0, 0)),
        pl.BlockSpec((RB2, 1), lambda i: (i, 0)),
    ],
    out_specs=pl.BlockSpec((RB2, D), lambda i: (i, 0)),
    out_shape=jax.ShapeDtypeStruct((TROWS, D), jnp.float32),
)


def _mid_body(p_ref, inv_ref, e2_ref):
    i = pl.program_id(0)
    en = p_ref[0] + p_ref[1]
    rows = i * RB2 + lax.broadcasted_iota(jnp.int32, (RB2, 1), 0)
    e2_ref[...] = jnp.where(rows < N, en * inv_ref[...], 0.0)


_mid = pl.pallas_call(
    _mid_body,
    grid=(5,),
    in_specs=[
        pl.BlockSpec((2, RB2, D), lambda i: (0, i, 0)),
        pl.BlockSpec((RB2, 1), lambda i: (i, 0)),
    ],
    out_specs=pl.BlockSpec((RB2, D), lambda i: (i, 0)),
    out_shape=jax.ShapeDtypeStruct((TROWS, D), jnp.float32),
)


def _eout_body(p_ref, we_ref, be_ref, eout_ref):
    en = p_ref[0] + p_ref[1]
    eout_ref[...] = (
        jnp.dot(en, we_ref[...], preferred_element_type=jnp.float32) + be_ref[...]
    )


_eout = pl.pallas_call(
    _eout_body,
    grid=(N // RB,),
    in_specs=[
        pl.BlockSpec((2, RB, D), lambda i: (0, i, 0)),
        pl.BlockSpec((D, D_E), lambda i: (0, 0)),
        pl.BlockSpec((1, D_E), lambda i: (0, 0)),
    ],
    out_specs=pl.BlockSpec((RB, D_E), lambda i: (i, 0)),
    out_shape=jax.ShapeDtypeStruct((N, D_E), jnp.float32),
)


def _final_body(s_ref, dv2_ref, out_ref):
    out_ref[...] = jnp.maximum((s_ref[0] + s_ref[1]) * dv2_ref[...], 0.0)


_final = pl.pallas_call(
    _final_body,
    grid=(N // RB,),
    in_specs=[
        pl.BlockSpec((2, RB, D), lambda i: (0, i, 0)),
        pl.BlockSpec((RB, 1), lambda i: (i, 0)),
    ],
    out_specs=pl.BlockSpec((RB, D), lambda i: (i, 0)),
    out_shape=jax.ShapeDtypeStruct((N, D), jnp.float32),
)


EPN = E // NW
EPT = NCHUNK * CH - EPN
TPW = NCHUNK * CH


def _pad_block():
    cols = lax.broadcasted_iota(jnp.int32, (NW, EPT), 1)
    wv = lax.broadcasted_iota(jnp.int32, (NW, EPT), 0)
    return (N + cols) | (((cols + wv * EPT) % N) << 16)


def _pack1_body(ei_ref, pk_ref):
    pk_ref[:, :EPN] = ei_ref[0] | (ei_ref[1] << 16)
    pk_ref[:, EPN:] = _pad_block()


def _pack2_body(ei_ref, pk_ref):
    pk_ref[:, :EPN] = ei_ref[1] | (ei_ref[0] << 16)
    pk_ref[:, EPN:] = _pad_block()


_pack1 = pl.pallas_call(
    _pack1_body, out_shape=jax.ShapeDtypeStruct((NW, TPW), jnp.int32)
)
_pack2 = pl.pallas_call(
    _pack2_body, out_shape=jax.ShapeDtypeStruct((NW, TPW), jnp.int32)
)


def _sc_body(
    table, pidx, out,
    pk_v, g0, g1, g2, g3, s0, s1, s2, s3, b0, b1, b2, b3, acc,
    gm0, gm1, gm2, gm3, sm0, sm1, sm2, sm3,
):
    gs = (g0, g1, g2, g3)
    ss = (s0, s1, s2, s3)
    bs = (b0, b1, b2, b3)
    gms = (gm0, gm1, gm2, gm3)
    sms = (sm0, sm1, sm2, sm3)
    cid = lax.axis_index("c")
    sid = lax.axis_index("s")
    w = sid * NC + cid
    start = pl.multiple_of(sid * RPT, 8)
    pltpu.sync_copy(pidx.at[w], pk_v)

    def unpack(row, half, gbuf, sbuf):
        for j in range(SCH // 16):
            v = pk_v[row, pl.ds(half * SCH + j * 16, 16)]
            gbuf[pl.ds(j * 16, 16)] = v & 0xFFFF
            sbuf[pl.ds(j * 16, 16)] = lax.shift_right_logical(v, 16)

    def zrow(i, carry):
        for j in range(D // 16):
            b0[i, pl.ds(j * 16, 16)] = jnp.zeros((16,), jnp.float32)
        return carry

    lax.fori_loop(0, SCH, zrow, 0)

    @pl.when(sid < NS - 1)
    def _():
        for k in range(RPT // SCH):
            pltpu.sync_copy(b0, acc.at[pl.ds(start + k * SCH, SCH), :])
        pltpu.sync_copy(
            b0.at[pl.ds(0, RPT % SCH)],
            acc.at[pl.ds(start + (RPT // SCH) * SCH, RPT % SCH), :],
        )

    @pl.when(sid == NS - 1)
    def _():
        for k in range(RPT_LAST // SCH):
            pltpu.sync_copy(b0, acc.at[pl.ds(start + k * SCH, SCH), :])

    plsc.subcore_barrier()

    for k in range(4):
        unpack(k // 2, k % 2, gs[k], ss[k])
        pltpu.async_copy(table.at[gs[k]], bs[k], gms[k])
        if k >= 1:
            pltpu.make_async_copy(table.at[gs[k - 1]], bs[k - 1], gms[k - 1]).wait()
            pltpu.async_copy(bs[k - 1], acc.at[ss[k - 1]], sms[k - 1], add=True)

    def group(g, carry):
        row = 2 * g
        for k in range(4):
            km1 = (k - 1) % 4
            pltpu.make_async_copy(bs[k], acc.at[ss[k]], sms[k]).wait()
            unpack(row + k // 2, k % 2, gs[k], ss[k])
            pltpu.async_copy(table.at[gs[k]], bs[k], gms[k])
            pltpu.make_async_copy(table.at[gs[km1]], bs[km1], gms[km1]).wait()
            pltpu.async_copy(bs[km1], acc.at[ss[km1]], sms[km1], add=True)
        return carry

    lax.fori_loop(1, NGRP, group, 0)
    pltpu.make_async_copy(table.at[gs[3]], bs[3], gms[3]).wait()
    pltpu.async_copy(bs[3], acc.at[ss[3]], sms[3], add=True)
    for k in range(4):
        pltpu.make_async_copy(bs[k], acc.at[ss[k]], sms[k]).wait()
    plsc.subcore_barrier()

    @pl.when(sid < NS - 1)
    def _():
        pltpu.sync_copy(
            acc.at[pl.ds(start, RPT), :], out.at[cid, pl.ds(start, RPT), :]
        )

    @pl.when(sid == NS - 1)
    def _():
        pltpu.sync_copy(
            acc.at[pl.ds(start, RPT_LAST), :],
            out.at[cid, pl.ds(start, RPT_LAST), :],
        )


_sc_scatter = pl.kernel(
    _sc_body,
    out_type=jax.ShapeDtypeStruct((NC, N, D), jnp.float32),
    mesh=plsc.VectorSubcoreMesh(
        core_axis_name="c", subcore_axis_name="s", num_cores=NC, num_subcores=NS
    ),
    scratch_types=(
        [pltpu.VMEM((NCHUNK, CH), jnp.int32)]
        + [pltpu.VMEM((SCH,), jnp.int32)] * 8
        + [pltpu.VMEM((SCH, D), jnp.float32)] * 4
        + [pltpu.VMEM_SHARED((N, D), jnp.float32)]
        + [pltpu.SemaphoreType.DMA] * 8
    ),
)


def kernel(vfeat, efeat, DV2, invDE, edge_index, W_v, b_v, W_e, b_e):
    src = edge_index[0]
    dst = edge_index[1]
    ei = edge_index.reshape(2, NW, EPN)
    pk1 = _pack1(ei).reshape(NW, NCHUNK, CH)
    pk2 = _pack2(ei).reshape(NW, NCHUNK, CH)
    dv2c = DV2.reshape(N, 1)
    t1 = _prep(vfeat, W_v, b_v.reshape(1, D), dv2c)
    p = _sc_scatter(t1, pk1)
    e2 = _mid(p, invDE.reshape(N, 1))
    s = _sc_scatter(e2, pk2)
    efeat_out = _eout(p, W_e, b_e.reshape(1, D_E))
    vfeat_out = _final(s, dv2c)
    return (vfeat_out, efeat_out)


# speedup vs baseline: 1.0528x; 1.0078x over previous
import jax
import jax.numpy as jnp
from jax import lax
from jax.experimental import pallas as pl
from jax.experimental.pallas import tpu as pltpu
from jax.experimental.pallas import tpu_sc as plsc

N = 10000
E = 320000
D = 128
D_E = 16
NC, NS = 2, 16
NW = NC * NS
CH = 128
NCHUNK = 80
SCH = 64
NSTREAM = NCHUNK * (CH // SCH)
NGRP = NSTREAM // 4
EPAD = NW * NCHUNK * CH - E
RPT = 624
RPT_LAST = N - RPT * (NS - 1)
RB = 2000
TEXTRA = 400
TROWS = N + TEXTRA
RB2 = TROWS // 5


def _prep_body(vfeat_ref, w_ref, b_ref, dv2_ref, out_ref):
    i = pl.program_id(0)
    wh = jnp.dot(vfeat_ref[...], w_ref[...], preferred_element_type=jnp.float32)
    rows = i * RB2 + lax.broadcasted_iota(jnp.int32, (RB2, 1), 0)
    out_ref[...] = jnp.where(rows < N, (wh + b_ref[...]) * dv2_ref[...], 0.0)


_prep = pl.pallas_call(
    _prep_body,
    grid=(5,),
    in_specs=[
        pl.BlockSpec((RB2, D), lambda i: (i, 0)),
        pl.BlockSpec((D, D), lambda i: (0, 0)),
        pl.BlockSpec((1, D), lambda i: (0, 0)),
        pl.BlockSpec((RB2, 1), lambda i: (i, 0)),
    ],
    out_specs=pl.BlockSpec((RB2, D), lambda i: (i, 0)),
    out_shape=jax.ShapeDtypeStruct((TROWS, D), jnp.float32),
)


def _mid_body(p_ref, inv_ref, e2_ref):
    i = pl.program_id(0)
    en = p_ref[0] + p_ref[1]
    rows = i * RB2 + lax.broadcasted_iota(jnp.int32, (RB2, 1), 0)
    e2_ref[...] = jnp.where(rows < N, en * inv_ref[...], 0.0)


_mid = pl.pallas_call(
    _mid_body,
    grid=(5,),
    in_specs=[
        pl.BlockSpec((2, RB2, D), lambda i: (0, i, 0)),
        pl.BlockSpec((RB2, 1), lambda i: (i, 0)),
    ],
    out_specs=pl.BlockSpec((RB2, D), lambda i: (i, 0)),
    out_shape=jax.ShapeDtypeStruct((TROWS, D), jnp.float32),
)


def _eout_body(p_ref, we_ref, be_ref, eout_ref):
    en = p_ref[0] + p_ref[1]
    eout_ref[...] = (
        jnp.dot(en, we_ref[...], preferred_element_type=jnp.float32) + be_ref[...]
    )


_eout = pl.pallas_call(
    _eout_body,
    grid=(N // RB,),
    in_specs=[
        pl.BlockSpec((2, RB, D), lambda i: (0, i, 0)),
        pl.BlockSpec((D, D_E), lambda i: (0, 0)),
        pl.BlockSpec((1, D_E), lambda i: (0, 0)),
    ],
    out_specs=pl.BlockSpec((RB, D_E), lambda i: (i, 0)),
    out_shape=jax.ShapeDtypeStruct((N, D_E), jnp.float32),
)


def _final_body(s_ref, dv2_ref, out_ref):
    out_ref[...] = jnp.maximum((s_ref[0] + s_ref[1]) * dv2_ref[...], 0.0)


_final = pl.pallas_call(
    _final_body,
    grid=(N // RB,),
    in_specs=[
        pl.BlockSpec((2, RB, D), lambda i: (0, i, 0)),
        pl.BlockSpec((RB, 1), lambda i: (i, 0)),
    ],
    out_specs=pl.BlockSpec((RB, D), lambda i: (i, 0)),
    out_shape=jax.ShapeDtypeStruct((N, D), jnp.float32),
)


EPN = E // NW
EPT = NCHUNK * CH - EPN
TPW = NCHUNK * CH


def _pad_block():
    cols = lax.broadcasted_iota(jnp.int32, (NW, EPT), 1)
    wv = lax.broadcasted_iota(jnp.int32, (NW, EPT), 0)
    return (N + cols) | (((cols + wv * EPT) % N) << 16)


def _pack1_body(ei_ref, pk_ref):
    pk_ref[:, :EPN] = ei_ref[0] | (ei_ref[1] << 16)
    pk_ref[:, EPN:] = _pad_block()


def _pack2_body(ei_ref, pk_ref):
    pk_ref[:, :EPN] = ei_ref[1] | (ei_ref[0] << 16)
    pk_ref[:, EPN:] = _pad_block()


_pack1 = pl.pallas_call(
    _pack1_body, out_shape=jax.ShapeDtypeStruct((NW, TPW), jnp.int32)
)
_pack2 = pl.pallas_call(
    _pack2_body, out_shape=jax.ShapeDtypeStruct((NW, TPW), jnp.int32)
)


def _sc_body(
    table, pidx, out,
    pk_v, g0, g1, g2, g3, s0, s1, s2, s3, b0, b1, b2, b3, acc,
    gm0, gm1, gm2, gm3, sm0, sm1, sm2, sm3,
):
    gs = (g0, g1, g2, g3)
    ss = (s0, s1, s2, s3)
    bs = (b0, b1, b2, b3)
    gms = (gm0, gm1, gm2, gm3)
    sms = (sm0, sm1, sm2, sm3)
    cid = lax.axis_index("c")
    sid = lax.axis_index("s")
    w = sid * NC + cid
    start = pl.multiple_of(sid * RPT, 8)
    pltpu.sync_copy(pidx.at[w], pk_v)

    def unpack(i, gbuf, sbuf):
        base = pl.multiple_of(i * SCH, 8)
        for j in range(SCH // 16):
            v = pk_v[pl.ds(base + j * 16, 16)]
            gbuf[pl.ds(j * 16, 16)] = v & 0xFFFF
            sbuf[pl.ds(j * 16, 16)] = lax.shift_right_logical(v, 16)

    def zrow(i, carry):
        for j in range(D // 16):
            b0[i, pl.ds(j * 16, 16)] = jnp.zeros((16,), jnp.float32)
        return carry

    lax.fori_loop(0, SCH, zrow, 0)

    @pl.when(sid < NS - 1)
    def _():
        for k in range(RPT // SCH):
            pltpu.sync_copy(b0, acc.at[pl.ds(start + k * SCH, SCH), :])
        pltpu.sync_copy(
            b0.at[pl.ds(0, RPT % SCH)],
            acc.at[pl.ds(start + (RPT // SCH) * SCH, RPT % SCH), :],
        )

    @pl.when(sid == NS - 1)
    def _():
        for k in range(RPT_LAST // SCH):
            pltpu.sync_copy(b0, acc.at[pl.ds(start + k * SCH, SCH), :])

    plsc.subcore_barrier()

    for k in range(4):
        unpack(k, gs[k], ss[k])
        pltpu.async_copy(table.at[gs[k]], bs[k], gms[k])
        if k >= 1:
            pltpu.make_async_copy(table.at[gs[k - 1]], bs[k - 1], gms[k - 1]).wait()
            pltpu.async_copy(bs[k - 1], acc.at[ss[k - 1]], sms[k - 1], add=True)

    def group(g, carry):
        for k in range(4):
            km1 = (k - 1) % 4
            pltpu.make_async_copy(bs[k], acc.at[ss[k]], sms[k]).wait()
            unpack(4 * g + k, gs[k], ss[k])
            pltpu.async_copy(table.at[gs[k]], bs[k], gms[k])
            pltpu.make_async_copy(table.at[gs[km1]], bs[km1], gms[km1]).wait()
            pltpu.async_copy(bs[km1], acc.at[ss[km1]], sms[km1], add=True)
        return carry

    lax.fori_loop(1, NGRP, group, 0)
    pltpu.make_async_copy(table.at[gs[3]], bs[3], gms[3]).wait()
    pltpu.async_copy(bs[3], acc.at[ss[3]], sms[3], add=True)
    for k in range(4):
        pltpu.make_async_copy(bs[k], acc.at[ss[k]], sms[k]).wait()
    plsc.subcore_barrier()

    @pl.when(sid < NS - 1)
    def _():
        pltpu.sync_copy(
            acc.at[pl.ds(start, RPT), :], out.at[cid, pl.ds(start, RPT), :]
        )

    @pl.when(sid == NS - 1)
    def _():
        pltpu.sync_copy(
            acc.at[pl.ds(start, RPT_LAST), :],
            out.at[cid, pl.ds(start, RPT_LAST), :],
        )


_sc_scatter = pl.kernel(
    _sc_body,
    out_type=jax.ShapeDtypeStruct((NC, N, D), jnp.float32),
    mesh=plsc.VectorSubcoreMesh(
        core_axis_name="c", subcore_axis_name="s", num_cores=NC, num_subcores=NS
    ),
    scratch_types=(
        [pltpu.VMEM((TPW,), jnp.int32)]
        + [pltpu.VMEM((SCH,), jnp.int32)] * 8
        + [pltpu.VMEM((SCH, D), jnp.float32)] * 4
        + [pltpu.VMEM_SHARED((N, D), jnp.float32)]
        + [pltpu.SemaphoreType.DMA] * 8
    ),
)


def kernel(vfeat, efeat, DV2, invDE, edge_index, W_v, b_v, W_e, b_e):
    src = edge_index[0]
    dst = edge_index[1]
    ei = edge_index.reshape(2, NW, EPN)
    pk1 = _pack1(ei)
    pk2 = _pack2(ei)
    dv2c = DV2.reshape(N, 1)
    t1 = _prep(vfeat, W_v, b_v.reshape(1, D), dv2c)
    p = _sc_scatter(t1, pk1)
    e2 = _mid(p, invDE.reshape(N, 1))
    s = _sc_scatter(e2, pk2)
    efeat_out = _eout(p, W_e, b_e.reshape(1, D_E))
    vfeat_out = _final(s, dv2c)
    return (vfeat_out, efeat_out)
